# baseline (speedup 1.0000x reference)
import jax
import jax.numpy as jnp
from jax.experimental import pallas as pl
from jax.experimental.pallas import tpu as pltpu


_NCHUNKS = 8


def _copy_body(x_hbm, y_hbm, ld_ref, sems):
    rows = x_hbm.shape[0]
    chunk = rows // _NCHUNKS
    copies = []
    for i in range(_NCHUNKS):
        c = pltpu.make_async_copy(
            x_hbm.at[pl.ds(i * chunk, chunk)],
            y_hbm.at[pl.ds(i * chunk, chunk)],
            sems.at[i],
        )
        c.start()
        copies.append(c)
    ld_ref[...] = jnp.zeros_like(ld_ref)
    for c in copies:
        c.wait()


def kernel(inputs):
    n, c = inputs.shape
    outputs, ld = pl.pallas_call(
        _copy_body,
        in_specs=[pl.BlockSpec(memory_space=pl.ANY)],
        out_specs=[
            pl.BlockSpec(memory_space=pl.ANY),
            pl.BlockSpec(memory_space=pltpu.VMEM),
        ],
        out_shape=[
            jax.ShapeDtypeStruct((n, c), inputs.dtype),
            jax.ShapeDtypeStruct((n,), jnp.float32),
        ],
        scratch_shapes=[pltpu.SemaphoreType.DMA((_NCHUNKS,))],
    )(inputs)
    return (outputs, ld)


# speedup vs baseline: 48.6127x; 48.6127x over previous
import jax
import jax.numpy as jnp
from jax.experimental import pallas as pl


_ROWS = 131072
_COLS = 256
_BLOCK_ROWS = 8192


def _copy_body(x_ref, y_ref, ld_ref):
    y_ref[...] = x_ref[...]
    ld_ref[...] = jnp.zeros_like(ld_ref)


def kernel(inputs):
    n, c = inputs.shape
    block_rows = _BLOCK_ROWS if n % _BLOCK_ROWS == 0 else n
    grid = (n // block_rows,)
    outputs, ld = pl.pallas_call(
        _copy_body,
        grid=grid,
        in_specs=[pl.BlockSpec((block_rows, c), lambda i: (i, 0))],
        out_specs=[
            pl.BlockSpec((block_rows, c), lambda i: (i, 0)),
            pl.BlockSpec((block_rows,), lambda i: (i,)),
        ],
        out_shape=[
            jax.ShapeDtypeStruct((n, c), inputs.dtype),
            jax.ShapeDtypeStruct((n,), jnp.float32),
        ],
    )(inputs)
    return (outputs, ld)
